# Initial kernel scaffold; baseline (speedup 1.0000x reference)
import jax
import jax.numpy as jnp
from jax.experimental import pallas as pl


def kernel(x, batch_size, conv1_w, conv1_b, conv2_w, conv2_b, gcn1_w, gcn1_b, gcn2_w, gcn2_b, fc_w, fc_b):
    raise NotImplementedError("write your pallas kernel here")


# speedup vs baseline: 89.9483x; 89.9483x over previous
import jax
import jax.numpy as jnp
from jax.experimental import pallas as pl
from jax.experimental.pallas import tpu as pltpu

_L = 4096
_F = 8
_H = 64
_NC = 2
_P = _L // 4
_G = 8

_ISQRT2 = 0.7071067811865476
_ISQRT3 = 0.5773502691896258


def _chain_gcn(hlin, bias):
    g = hlin.shape[0]
    pos = jax.lax.broadcasted_iota(jnp.int32, (1, _P, 1), 1)
    dinv = jnp.where((pos == 0) | (pos == _P - 1), _ISQRT2, _ISQRT3).astype(
        jnp.float32)
    u = hlin * dinv
    z = jnp.zeros((g, 1, _H), jnp.float32)
    ul = jnp.concatenate([z, u[:, :-1, :]], axis=1)
    ur = jnp.concatenate([u[:, 1:, :], z], axis=1)
    return dinv * (ul + u + ur) + bias[None, :, :]


def _model_block(x_ref, c1w_ref, c1b_ref, c2w_ref, c2b_ref,
                 g1w_ref, g1b_ref, g2w_ref, g2b_ref,
                 fcw_ref, fcb_ref, out_ref):
    f32 = jnp.float32
    xr = x_ref[...]
    g = xr.shape[0]
    x0 = xr[:, 0 * _P:1 * _P]
    x1 = xr[:, 1 * _P:2 * _P]
    x2 = xr[:, 2 * _P:3 * _P]
    x3 = xr[:, 3 * _P:4 * _P]
    zc = jnp.zeros((g, 1), f32)
    x3m = jnp.concatenate([zc, x3[:, :-1]], axis=1)
    x0p = jnp.concatenate([x0[:, 1:], zc], axis=1)

    c1w = c1w_ref[...]
    w0 = c1w[:, 0:1, None]
    w1 = c1w[:, 1:2, None]
    w2 = c1w[:, 2:3, None]
    b1 = c1b_ref[...][:, :, None]
    y0 = w0 * x3m[None] + w1 * x0[None] + w2 * x1[None]
    y1 = w0 * x0[None] + w1 * x1[None] + w2 * x2[None]
    y2 = w0 * x1[None] + w1 * x2[None] + w2 * x3[None]
    y3 = w0 * x2[None] + w1 * x3[None] + w2 * x0p[None]
    p0 = jnp.maximum(jnp.maximum(y0, y1) + b1, 0.0)
    p1 = jnp.maximum(jnp.maximum(y2, y3) + b1, 0.0)

    z2 = jnp.zeros((_F, g, 1), f32)
    p1m = jnp.concatenate([z2, p1[:, :, :-1]], axis=2)
    p0p = jnp.concatenate([p0[:, :, 1:], z2], axis=2)
    c2w = c2w_ref[...]
    qe = None
    qo = None
    for i in range(_F):
        v0 = c2w[:, i, 0][:, None, None]
        v1 = c2w[:, i, 1][:, None, None]
        v2 = c2w[:, i, 2][:, None, None]
        te = v0 * p1m[i][None] + v1 * p0[i][None] + v2 * p1[i][None]
        to = v0 * p0[i][None] + v1 * p1[i][None] + v2 * p0p[i][None]
        qe = te if qe is None else qe + te
        qo = to if qo is None else qo + to
    b2 = c2b_ref[...][:, :, None]
    h2 = jnp.maximum(jnp.maximum(qe, qo) + b2, 0.0)

    nodes = jnp.transpose(h2, (1, 2, 0)).reshape(g * _P, _F)

    g1 = jnp.dot(nodes, g1w_ref[...], preferred_element_type=f32)
    a1 = jnp.maximum(_chain_gcn(g1.reshape(g, _P, _H), g1b_ref[...]), 0.0)

    g2 = jnp.dot(a1.reshape(g * _P, _H), g2w_ref[...],
                 preferred_element_type=f32)
    a2 = jnp.maximum(_chain_gcn(g2.reshape(g, _P, _H), g2b_ref[...]), 0.0)

    pooled = jnp.sum(a2, axis=1) * (1.0 / _P)
    out = jnp.dot(pooled, fcw_ref[...],
                  preferred_element_type=f32) + fcb_ref[...]
    out_ref[...] = out


def kernel(x, batch_size, conv1_w, conv1_b, conv2_w, conv2_b,
           gcn1_w, gcn1_b, gcn2_w, gcn2_b, fc_w, fc_b):
    b = x.shape[0]
    x2 = (x.reshape(b, _P, 4).transpose(0, 2, 1).reshape(b, _L)
          .astype(jnp.float32))
    c1w = conv1_w.reshape(_F, 3)
    c1b = conv1_b.reshape(_F, 1)
    c2b = conv2_b.reshape(_F, 1)
    g1b = gcn1_b.reshape(1, _H)
    g2b = gcn2_b.reshape(1, _H)
    fcb = fc_b.reshape(1, _NC)

    grid = (b // _G,)
    full = lambda shape: pl.BlockSpec(shape, lambda i: tuple(0 for _ in shape))
    out = pl.pallas_call(
        _model_block,
        grid=grid,
        in_specs=[
            pl.BlockSpec((_G, _L), lambda i: (i, 0)),
            full((_F, 3)),
            full((_F, 1)),
            full((_F, _F, 3)),
            full((_F, 1)),
            full((_F, _H)),
            full((1, _H)),
            full((_H, _H)),
            full((1, _H)),
            full((_H, _NC)),
            full((1, _NC)),
        ],
        out_specs=pl.BlockSpec((_G, _NC), lambda i: (i, 0)),
        out_shape=jax.ShapeDtypeStruct((b, _NC), jnp.float32),
        compiler_params=pltpu.CompilerParams(
            dimension_semantics=("parallel",),
        ),
    )(x2, c1w, c1b, conv2_w, c2b, gcn1_w, g1b, gcn2_w, g2b, fc_w, fcb)
    return out


# speedup vs baseline: 182.1901x; 2.0255x over previous
import jax
import jax.numpy as jnp
from jax.experimental import pallas as pl
from jax.experimental.pallas import tpu as pltpu

_L = 4096
_F = 8
_H = 64
_NC = 2
_P = _L // 4
_G = 16
_GH = _G // 2
_GP = _G * _P
_GPH = _GH * _P

_ISQRT2 = 0.7071067811865476
_ISQRT3 = 0.5773502691896258


def _model_block(x_ref, c1_ref, c1b_ref, c2_ref, c2b_ref,
                 g1w_ref, g1b_ref, g2w_ref, g2b_ref,
                 fcw_ref, fcb_ref, out_ref):
    f32 = jnp.float32
    pos = jax.lax.broadcasted_iota(jnp.int32, (1, _GPH), 1) % _P
    first = pos == 0
    last = pos == _P - 1
    dinv = jnp.where(first | last, _ISQRT2, _ISQRT3).astype(f32)

    def agg(u):
        ul = jnp.where(first, 0.0, jnp.roll(u, 1, axis=1))
        ur = jnp.where(last, 0.0, jnp.roll(u, -1, axis=1))
        return ul + u + ur

    def half(xp):
        x3m = jnp.where(first, 0.0, jnp.roll(xp[3:4], 1, axis=1))
        x0p = jnp.where(last, 0.0, jnp.roll(xp[0:1], -1, axis=1))
        x6 = jnp.concatenate([x3m, xp, x0p], axis=0)
        y = jnp.dot(c1_ref[...], x6, preferred_element_type=f32)
        b1 = c1b_ref[...]
        p0 = jnp.maximum(jnp.maximum(y[0:8], y[8:16]) + b1, 0.0)
        p1 = jnp.maximum(jnp.maximum(y[16:24], y[24:32]) + b1, 0.0)

        p1m = jnp.where(first, 0.0, jnp.roll(p1, 1, axis=1))
        p0p = jnp.where(last, 0.0, jnp.roll(p0, -1, axis=1))
        ps = jnp.concatenate([p1m, p0, p1, p0p], axis=0)
        q = jnp.dot(c2_ref[...], ps, preferred_element_type=f32)
        h2 = jnp.maximum(
            jnp.maximum(q[0:8], q[8:16]) + c2b_ref[...], 0.0)

        s1 = dinv * agg(h2 * dinv)
        g1 = jnp.dot(g1w_ref[...], s1, preferred_element_type=f32)
        a1 = jnp.maximum(g1 + g1b_ref[...], 0.0)

        s2 = dinv * agg(a1 * dinv)
        g2 = jnp.dot(g2w_ref[...], s2, preferred_element_type=f32)
        a2 = jnp.maximum(g2 + g2b_ref[...], 0.0)

        cols = [jnp.sum(a2[:, i * _P:(i + 1) * _P], axis=1, keepdims=True)
                for i in range(_GH)]
        return jnp.concatenate(cols, axis=1)

    xall = x_ref[...]
    pa = half(xall[:, :_GPH])
    pb = half(xall[:, _GPH:])
    pooled = jnp.concatenate([pa, pb], axis=1) * (1.0 / _P)
    out = jnp.dot(jnp.transpose(pooled), fcw_ref[...],
                  preferred_element_type=f32) + fcb_ref[...]
    out_ref[...] = out


def kernel(x, batch_size, conv1_w, conv1_b, conv2_w, conv2_b,
           gcn1_w, gcn1_b, gcn2_w, gcn2_b, fc_w, fc_b):
    b = x.shape[0]
    f32 = jnp.float32
    xp = (x.reshape(b, _P, 4).transpose(2, 0, 1).reshape(4, b * _P)
          .astype(f32))

    c1w = conv1_w.reshape(_F, 3).astype(f32)
    c1 = jnp.concatenate(
        [jnp.concatenate(
            [jnp.zeros((_F, r), f32), c1w, jnp.zeros((_F, 3 - r), f32)],
            axis=1) for r in range(4)],
        axis=0)
    w0, w1, w2 = (conv2_w[:, :, 0].astype(f32), conv2_w[:, :, 1].astype(f32),
                  conv2_w[:, :, 2].astype(f32))
    z8 = jnp.zeros((_F, _F), f32)
    c2 = jnp.concatenate(
        [jnp.concatenate([w0, w1, w2, z8], axis=1),
         jnp.concatenate([z8, w0, w1, w2], axis=1)],
        axis=0)

    args = (
        xp,
        c1,
        conv1_b.reshape(_F, 1).astype(f32),
        c2,
        conv2_b.reshape(_F, 1).astype(f32),
        gcn1_w.T.astype(f32),
        gcn1_b.reshape(_H, 1).astype(f32),
        gcn2_w.T.astype(f32),
        gcn2_b.reshape(_H, 1).astype(f32),
        fc_w.astype(f32),
        fc_b.reshape(1, _NC).astype(f32),
    )
    grid = (b * _P // _GP,)
    full = lambda shape: pl.BlockSpec(shape, lambda i: tuple(0 for _ in shape))
    out = pl.pallas_call(
        _model_block,
        grid=grid,
        in_specs=[
            pl.BlockSpec((4, _GP), lambda i: (0, i)),
            full((4 * _F, 6)),
            full((_F, 1)),
            full((2 * _F, 4 * _F)),
            full((_F, 1)),
            full((_H, _F)),
            full((_H, 1)),
            full((_H, _H)),
            full((_H, 1)),
            full((_H, _NC)),
            full((1, _NC)),
        ],
        out_specs=pl.BlockSpec((_G, _NC), lambda i: (i, 0)),
        out_shape=jax.ShapeDtypeStruct((b, _NC), f32),
        compiler_params=pltpu.CompilerParams(
            dimension_semantics=("parallel",),
        ),
    )(*args)
    return out
